# baseline (speedup 1.0000x reference)
import functools
import math

import jax
import jax.numpy as jnp
from jax import lax
from jax.experimental import pallas as pl
from jax.experimental.pallas import tpu as pltpu
from jax.experimental.pallas import tpu_sc as plsc

_B = 4096
_NW = 32
_RPW = _B // _NW
_NLIST = 50
_NCAT = 13
_NCONT = 8
_NEMB = 64
_NDENSE = _NCONT + _NEMB
_BN_SCALE = (1.0 + 1e-3) ** -0.5
_LANE = 16
_CHUNKS = _RPW // _LANE


def _body_list(t_l, il_hbm, out_hbm, idxl_v, vl_v, out_v, sem_l, sem_g):
    wid = lax.axis_index("s") * 2 + lax.axis_index("c")
    base = wid * _RPW

    pltpu.async_copy(il_hbm.at[wid], idxl_v, sem_l).wait()

    def fire(j, carry):
        pltpu.async_copy(t_l.at[idxl_v.at[j]], vl_v.at[j], sem_g)
        return carry

    lax.fori_loop(0, _NLIST, fire, 0)

    def drain_body(j, carry):
        pltpu.make_async_copy(t_l.at[idxl_v.at[0]], vl_v.at[0],
                              sem_g).wait()
        return carry

    lax.fori_loop(0, _NLIST, drain_body, 0)

    zeros = tuple(jnp.zeros((_LANE,), jnp.float32) for _ in range(_CHUNKS))

    def red_body(j, accs):
        return tuple(accs[k] + vl_v[j, pl.ds(k * _LANE, _LANE)]
                     for k in range(_CHUNKS))

    accs = lax.fori_loop(0, _NLIST, red_body, zeros)
    for k in range(_CHUNKS):
        out_v[pl.ds(k * _LANE, _LANE)] = accs[k]

    pltpu.sync_copy(out_v, out_hbm.at[pl.ds(base, _RPW)])


def _body_rest(t_c0, t_c1, t_c2, t_c3, t_c4, t_c5, t_c6, t_c7, t_c8,
               t_c9, t_c10, t_c11, t_c12,
               s0_hbm, s1_hbm, s2_hbm, s3_hbm, s4_hbm, s5_hbm, s6_hbm,
               s7_hbm, s8_hbm, s9_hbm, s10_hbm, s11_hbm, s12_hbm,
               n0_hbm, n1_hbm, n2_hbm, n3_hbm, n4_hbm, n5_hbm, n6_hbm,
               n7_hbm, ce_hbm, w_hbm, out_hbm,
               idxc_v, vs_v, cont_v, ce_v, w_v, out_v,
               sem_idxc, sem_dense, sem_g):
    wid = lax.axis_index("s") * 2 + lax.axis_index("c")
    base = wid * _RPW
    rows = pl.ds(base, _RPW)

    singles = (s0_hbm, s1_hbm, s2_hbm, s3_hbm, s4_hbm, s5_hbm, s6_hbm,
               s7_hbm, s8_hbm, s9_hbm, s10_hbm, s11_hbm, s12_hbm)
    cp_idxc = [pltpu.async_copy(s.at[rows], idxc_v.at[i], sem_idxc)
               for i, s in enumerate(singles)]
    conts = (n0_hbm, n1_hbm, n2_hbm, n3_hbm, n4_hbm, n5_hbm, n6_hbm, n7_hbm)
    cp_dense = [pltpu.async_copy(c.at[rows], cont_v.at[i], sem_dense)
                for i, c in enumerate(conts)]
    cp_dense.append(pltpu.async_copy(ce_hbm.at[wid], ce_v, sem_dense))
    cp_dense.append(pltpu.async_copy(w_hbm, w_v, sem_dense))

    for cp in cp_idxc:
        cp.wait()
    for i, t in enumerate((t_c0, t_c1, t_c2, t_c3, t_c4, t_c5, t_c6, t_c7,
                           t_c8, t_c9, t_c10, t_c11, t_c12)):
        pltpu.async_copy(t.at[idxc_v.at[i]], vs_v.at[i], sem_g)

    for cp in cp_dense:
        cp.wait()

    zeros = tuple(jnp.zeros((_LANE,), jnp.float32) for _ in range(_CHUNKS))

    def cont_body(j, accs):
        w_b = w_v[j, pl.ds(0, _LANE)]
        return tuple(accs[k] + cont_v[j, pl.ds(k * _LANE, _LANE)] * w_b
                     for k in range(_CHUNKS))

    dense_accs = lax.fori_loop(0, _NCONT, cont_body, zeros)

    def emb_body(j, accs):
        w_b = w_v[_NCONT + j, pl.ds(0, _LANE)]
        return tuple(accs[k] + ce_v[j, pl.ds(k * _LANE, _LANE)] * w_b
                     for k in range(_CHUNKS))

    dense_accs = lax.fori_loop(0, _NEMB, emb_body, dense_accs)

    def drain_gs(j, carry):
        pltpu.make_async_copy(t_c0.at[idxc_v.at[0]], vs_v.at[0],
                              sem_g).wait()
        return carry

    lax.fori_loop(0, _NCAT, drain_gs, 0)

    def red_body(j, accs):
        return tuple(accs[k] + vs_v[j, pl.ds(k * _LANE, _LANE)]
                     for k in range(_CHUNKS))

    cat_accs = lax.fori_loop(0, _NCAT, red_body, zeros)

    bias = (w_v[_NDENSE, pl.ds(0, _LANE)]
            + w_v[_NDENSE + 1, pl.ds(0, _LANE)])
    for k in range(_CHUNKS):
        out_v[pl.ds(k * _LANE, _LANE)] = (
            cat_accs[k] + _BN_SCALE * dense_accs[k] + bias)

    pltpu.sync_copy(out_v, out_hbm.at[pl.ds(base, _RPW)])


@jax.jit
def _run(tables, il0, il1, singles, conts, ce, w_all):
    mesh = plsc.VectorSubcoreMesh(core_axis_name="c", subcore_axis_name="s")
    k_list = pl.kernel(
        _body_list,
        mesh=mesh,
        out_type=jax.ShapeDtypeStruct((_B,), jnp.float32),
        scratch_types=[
            pltpu.VMEM((_NLIST, _RPW), jnp.int32),
            pltpu.VMEM((_NLIST, _RPW), jnp.float32),
            pltpu.VMEM((_RPW,), jnp.float32),
            pltpu.SemaphoreType.DMA,
            pltpu.SemaphoreType.DMA,
        ],
    )
    k_rest = pl.kernel(
        _body_rest,
        mesh=mesh,
        out_type=jax.ShapeDtypeStruct((_B,), jnp.float32),
        scratch_types=[
            pltpu.VMEM((_NCAT, _RPW), jnp.int32),
            pltpu.VMEM((_NCAT, _RPW), jnp.float32),
            pltpu.VMEM((_NCONT, _RPW), jnp.float32),
            pltpu.VMEM((_NEMB, _RPW), jnp.float32),
            pltpu.VMEM((80, _LANE), jnp.float32),
            pltpu.VMEM((_RPW,), jnp.float32),
            pltpu.SemaphoreType.DMA,
            pltpu.SemaphoreType.DMA,
            pltpu.SemaphoreType.DMA,
        ],
    )
    part0 = k_list(tables[0], il0)
    part1 = k_list(tables[1], il1)
    part2 = k_rest(*tables[2:], *singles, *conts, ce, w_all)
    return part0 + part1 + part2


def kernel(catlist_0, catlist_1, cat_0, cat_1, cat_2, cat_3, cat_4, cat_5,
           cat_6, cat_7, cat_8, cat_9, cat_10, cat_11, cat_12,
           cont_0, cont_1, cont_2, cont_3, cont_4, cont_5, cont_6, cont_7,
           contembd_0, contembd_1, contembd_2, contembd_3,
           table_catlist_0, table_catlist_1,
           table_cat_0, table_cat_1, table_cat_2, table_cat_3, table_cat_4,
           table_cat_5, table_cat_6, table_cat_7, table_cat_8, table_cat_9,
           table_cat_10, table_cat_11, table_cat_12,
           W1, b1, W2, b2):
    tables = [t.reshape(-1) for t in
              (table_catlist_0, table_catlist_1,
               table_cat_0, table_cat_1, table_cat_2, table_cat_3,
               table_cat_4, table_cat_5, table_cat_6, table_cat_7,
               table_cat_8, table_cat_9, table_cat_10, table_cat_11,
               table_cat_12)]
    singles = [c.reshape(-1) for c in
               (cat_0, cat_1, cat_2, cat_3, cat_4, cat_5, cat_6, cat_7,
                cat_8, cat_9, cat_10, cat_11, cat_12)]
    conts = [c.reshape(-1) for c in
             (cont_0, cont_1, cont_2, cont_3, cont_4, cont_5, cont_6,
              cont_7)]

    il0 = catlist_0.reshape(_NW, _RPW, _NLIST).transpose(0, 2, 1)
    il1 = catlist_1.reshape(_NW, _RPW, _NLIST).transpose(0, 2, 1)
    ce = jnp.concatenate(
        [contembd_0, contembd_1, contembd_2, contembd_3], axis=1)
    ce = ce.reshape(_NW, _RPW, _NEMB).transpose(0, 2, 1)

    w_all = jnp.concatenate(
        [W1.reshape(-1), W2.reshape(-1), b1.reshape(-1), b2.reshape(-1),
         jnp.zeros((6,), jnp.float32)])
    w_all = jnp.tile(w_all[:, None], (1, _LANE))

    out = _run(tables, il0, il1, singles, conts, ce, w_all)
    return out.reshape(_B, 1)


# speedup vs baseline: 1.3278x; 1.3278x over previous
import functools
import math

import jax
import jax.numpy as jnp
from jax import lax
from jax.experimental import pallas as pl
from jax.experimental.pallas import tpu as pltpu
from jax.experimental.pallas import tpu_sc as plsc

_B = 4096
_NW = 32
_RPW = _B // _NW
_NLIST = 50
_NCAT = 13
_NCONT = 8
_NEMB = 64
_NDENSE = _NCONT + _NEMB
_BN_SCALE = (1.0 + 1e-3) ** -0.5
_LANE = 16
_CHUNKS = _RPW // _LANE


def _body_lists(t_l0, t_l1, il0_hbm, il1_hbm, out_hbm,
                idxl0_v, idxl1_v, vl0_v, vl1_v, out_v,
                sem_l0, sem_l1, sem_g):
    wid = lax.axis_index("s") * 2 + lax.axis_index("c")
    base = wid * _RPW

    cp_l0 = pltpu.async_copy(il0_hbm.at[wid], idxl0_v, sem_l0)
    cp_l1 = pltpu.async_copy(il1_hbm.at[wid], idxl1_v, sem_l1)

    cp_l0.wait()

    def fire_l0(j, carry):
        pltpu.async_copy(t_l0.at[idxl0_v.at[j]], vl0_v.at[j], sem_g)
        return carry

    lax.fori_loop(0, _NLIST, fire_l0, 0)
    cp_l1.wait()

    def fire_l1(j, carry):
        pltpu.async_copy(t_l1.at[idxl1_v.at[j]], vl1_v.at[j], sem_g)
        return carry

    lax.fori_loop(0, _NLIST, fire_l1, 0)

    def drain_body(j, carry):
        pltpu.make_async_copy(t_l0.at[idxl0_v.at[0]], vl0_v.at[0],
                              sem_g).wait()
        return carry

    lax.fori_loop(0, 2 * _NLIST, drain_body, 0)

    zeros = tuple(jnp.zeros((_LANE,), jnp.float32) for _ in range(_CHUNKS))

    def red_body(j, accs):
        return tuple(accs[k] + vl0_v[j, pl.ds(k * _LANE, _LANE)]
                     + vl1_v[j, pl.ds(k * _LANE, _LANE)]
                     for k in range(_CHUNKS))

    accs = lax.fori_loop(0, _NLIST, red_body, zeros)
    for k in range(_CHUNKS):
        out_v[pl.ds(k * _LANE, _LANE)] = accs[k]

    pltpu.sync_copy(out_v, out_hbm.at[pl.ds(base, _RPW)])


def _body_rest(t_c0, t_c1, t_c2, t_c3, t_c4, t_c5, t_c6, t_c7, t_c8,
               t_c9, t_c10, t_c11, t_c12,
               s0_hbm, s1_hbm, s2_hbm, s3_hbm, s4_hbm, s5_hbm, s6_hbm,
               s7_hbm, s8_hbm, s9_hbm, s10_hbm, s11_hbm, s12_hbm,
               n0_hbm, n1_hbm, n2_hbm, n3_hbm, n4_hbm, n5_hbm, n6_hbm,
               n7_hbm, ce_hbm, w_hbm, out_hbm,
               idxc_v, vs_v, cont_v, ce_v, w_v, out_v,
               sem_idxc, sem_dense, sem_g):
    wid = lax.axis_index("s") * 2 + lax.axis_index("c")
    base = wid * _RPW
    rows = pl.ds(base, _RPW)

    singles = (s0_hbm, s1_hbm, s2_hbm, s3_hbm, s4_hbm, s5_hbm, s6_hbm,
               s7_hbm, s8_hbm, s9_hbm, s10_hbm, s11_hbm, s12_hbm)
    cp_idxc = [pltpu.async_copy(s.at[rows], idxc_v.at[i], sem_idxc)
               for i, s in enumerate(singles)]
    conts = (n0_hbm, n1_hbm, n2_hbm, n3_hbm, n4_hbm, n5_hbm, n6_hbm, n7_hbm)
    cp_dense = [pltpu.async_copy(c.at[rows], cont_v.at[i], sem_dense)
                for i, c in enumerate(conts)]
    cp_dense.append(pltpu.async_copy(ce_hbm.at[wid], ce_v, sem_dense))
    cp_dense.append(pltpu.async_copy(w_hbm, w_v, sem_dense))

    for cp in cp_idxc:
        cp.wait()
    for i, t in enumerate((t_c0, t_c1, t_c2, t_c3, t_c4, t_c5, t_c6, t_c7,
                           t_c8, t_c9, t_c10, t_c11, t_c12)):
        pltpu.async_copy(t.at[idxc_v.at[i]], vs_v.at[i], sem_g)

    for cp in cp_dense:
        cp.wait()

    zeros = tuple(jnp.zeros((_LANE,), jnp.float32) for _ in range(_CHUNKS))

    def cont_body(j, accs):
        w_b = w_v[j, pl.ds(0, _LANE)]
        return tuple(accs[k] + cont_v[j, pl.ds(k * _LANE, _LANE)] * w_b
                     for k in range(_CHUNKS))

    dense_accs = lax.fori_loop(0, _NCONT, cont_body, zeros)

    def emb_body(j, accs):
        w_b = w_v[_NCONT + j, pl.ds(0, _LANE)]
        return tuple(accs[k] + ce_v[j, pl.ds(k * _LANE, _LANE)] * w_b
                     for k in range(_CHUNKS))

    dense_accs = lax.fori_loop(0, _NEMB, emb_body, dense_accs)

    def drain_gs(j, carry):
        pltpu.make_async_copy(t_c0.at[idxc_v.at[0]], vs_v.at[0],
                              sem_g).wait()
        return carry

    lax.fori_loop(0, _NCAT, drain_gs, 0)

    def red_body(j, accs):
        return tuple(accs[k] + vs_v[j, pl.ds(k * _LANE, _LANE)]
                     for k in range(_CHUNKS))

    cat_accs = lax.fori_loop(0, _NCAT, red_body, zeros)

    bias = (w_v[_NDENSE, pl.ds(0, _LANE)]
            + w_v[_NDENSE + 1, pl.ds(0, _LANE)])
    for k in range(_CHUNKS):
        out_v[pl.ds(k * _LANE, _LANE)] = (
            cat_accs[k] + _BN_SCALE * dense_accs[k] + bias)

    pltpu.sync_copy(out_v, out_hbm.at[pl.ds(base, _RPW)])


@jax.jit
def _run(tables, il0, il1, singles, conts, ce, w_all):
    mesh = plsc.VectorSubcoreMesh(core_axis_name="c", subcore_axis_name="s")
    k_lists = pl.kernel(
        _body_lists,
        mesh=mesh,
        out_type=jax.ShapeDtypeStruct((_B,), jnp.float32),
        scratch_types=[
            pltpu.VMEM((_NLIST, _RPW), jnp.int32),
            pltpu.VMEM((_NLIST, _RPW), jnp.int32),
            pltpu.VMEM((_NLIST, _RPW), jnp.float32),
            pltpu.VMEM((_NLIST, _RPW), jnp.float32),
            pltpu.VMEM((_RPW,), jnp.float32),
            pltpu.SemaphoreType.DMA,
            pltpu.SemaphoreType.DMA,
            pltpu.SemaphoreType.DMA,
        ],
    )
    k_rest = pl.kernel(
        _body_rest,
        mesh=mesh,
        out_type=jax.ShapeDtypeStruct((_B,), jnp.float32),
        scratch_types=[
            pltpu.VMEM((_NCAT, _RPW), jnp.int32),
            pltpu.VMEM((_NCAT, _RPW), jnp.float32),
            pltpu.VMEM((_NCONT, _RPW), jnp.float32),
            pltpu.VMEM((_NEMB, _RPW), jnp.float32),
            pltpu.VMEM((80, _LANE), jnp.float32),
            pltpu.VMEM((_RPW,), jnp.float32),
            pltpu.SemaphoreType.DMA,
            pltpu.SemaphoreType.DMA,
            pltpu.SemaphoreType.DMA,
        ],
    )
    part1 = k_lists(tables[0], tables[1], il0, il1)
    part2 = k_rest(*tables[2:], *singles, *conts, ce, w_all)
    return part1 + part2


def kernel(catlist_0, catlist_1, cat_0, cat_1, cat_2, cat_3, cat_4, cat_5,
           cat_6, cat_7, cat_8, cat_9, cat_10, cat_11, cat_12,
           cont_0, cont_1, cont_2, cont_3, cont_4, cont_5, cont_6, cont_7,
           contembd_0, contembd_1, contembd_2, contembd_3,
           table_catlist_0, table_catlist_1,
           table_cat_0, table_cat_1, table_cat_2, table_cat_3, table_cat_4,
           table_cat_5, table_cat_6, table_cat_7, table_cat_8, table_cat_9,
           table_cat_10, table_cat_11, table_cat_12,
           W1, b1, W2, b2):
    tables = [t.reshape(-1) for t in
              (table_catlist_0, table_catlist_1,
               table_cat_0, table_cat_1, table_cat_2, table_cat_3,
               table_cat_4, table_cat_5, table_cat_6, table_cat_7,
               table_cat_8, table_cat_9, table_cat_10, table_cat_11,
               table_cat_12)]
    singles = [c.reshape(-1) for c in
               (cat_0, cat_1, cat_2, cat_3, cat_4, cat_5, cat_6, cat_7,
                cat_8, cat_9, cat_10, cat_11, cat_12)]
    conts = [c.reshape(-1) for c in
             (cont_0, cont_1, cont_2, cont_3, cont_4, cont_5, cont_6,
              cont_7)]

    il0 = catlist_0.reshape(_NW, _RPW, _NLIST).transpose(0, 2, 1)
    il1 = catlist_1.reshape(_NW, _RPW, _NLIST).transpose(0, 2, 1)
    ce = jnp.concatenate(
        [contembd_0, contembd_1, contembd_2, contembd_3], axis=1)
    ce = ce.reshape(_NW, _RPW, _NEMB).transpose(0, 2, 1)

    w_all = jnp.concatenate(
        [W1.reshape(-1), W2.reshape(-1), b1.reshape(-1), b2.reshape(-1),
         jnp.zeros((6,), jnp.float32)])
    w_all = jnp.tile(w_all[:, None], (1, _LANE))

    out = _run(tables, il0, il1, singles, conts, ce, w_all)
    return out.reshape(_B, 1)
